# baseline (speedup 1.0000x reference)
import jax
import jax.numpy as jnp
from jax import lax
from jax.experimental import pallas as pl
from jax.experimental.pallas import tpu as pltpu


def _make_fused_kernel(H, W, Cin, Cmid):
    HW = H * W

    def body(x_ref, w1_ref, s1_ref, b1_ref, w2_ref, b2_ref, out_ref):
        x = x_ref[0].astype(jnp.bfloat16)

        zrow = jnp.zeros((W, Cin), jnp.bfloat16)
        up = jnp.concatenate([zrow, x[:-W]], axis=0)
        dn = jnp.concatenate([x[W:], zrow], axis=0)

        xt = jnp.concatenate([up, x, dn], axis=1)
        wt = jnp.concatenate([w1_ref[0].astype(jnp.bfloat16),
                              w1_ref[1].astype(jnp.bfloat16),
                              w1_ref[2].astype(jnp.bfloat16)], axis=1)
        y = lax.dot_general(wt, xt, (((1,), (1,)), ((), ())),
                            preferred_element_type=jnp.float32)

        xx = lax.broadcasted_iota(jnp.int32, (1, HW), 1) % W
        ok_l = xx >= 1
        ok_r = xx <= W - 2

        mid = y[Cmid:2 * Cmid]
        lft = pltpu.roll(y[:Cmid], 1, 1)
        rgt = pltpu.roll(y[2 * Cmid:], HW - 1, 1)
        acc = (mid + jnp.where(ok_l, lft, 0.0)
               + jnp.where(ok_r, rgt, 0.0))

        s1 = s1_ref[:, :1]
        b1 = b1_ref[:, :1]
        h = jnp.maximum(acc * s1 + b1, 0.0)

        out = jnp.dot(w2_ref[...], h, preferred_element_type=jnp.float32)
        out = out + b2_ref[...]
        out_ref[...] = out.reshape(1, H, W)

    return body


def kernel(x_nchw, w1, b1_conv, gamma, beta, mean, var, eps, w2, b2):
    N, Cin, H, W = x_nchw.shape
    Cmid = w1.shape[0]
    Cout = w2.shape[0]
    HW = H * W

    x_t = jnp.transpose(x_nchw, (0, 2, 3, 1)).reshape(N, HW, Cin)

    w1_k = jnp.transpose(w1, (2, 3, 0, 1)).reshape(3, 3 * Cmid, Cin)

    scale = gamma / jnp.sqrt(var + eps)
    bias = (b1_conv - mean) * scale + beta
    s1 = jnp.broadcast_to(scale.reshape(Cmid, 1), (Cmid, 128))
    b1 = jnp.broadcast_to(bias.reshape(Cmid, 1), (Cmid, 128))

    w2_k = w2[:, :, 0, 0].astype(jnp.float32)
    b2_k = b2.reshape(Cout, 1).astype(jnp.float32)

    out = pl.pallas_call(
        _make_fused_kernel(H, W, Cin, Cmid),
        out_shape=jax.ShapeDtypeStruct((N, H, W), jnp.float32),
        grid=(N,),
        in_specs=[
            pl.BlockSpec((1, HW, Cin), lambda n: (n, 0, 0)),
            pl.BlockSpec((3, 3 * Cmid, Cin), lambda n: (0, 0, 0)),
            pl.BlockSpec((Cmid, 128), lambda n: (0, 0)),
            pl.BlockSpec((Cmid, 128), lambda n: (0, 0)),
            pl.BlockSpec((Cout, Cmid), lambda n: (0, 0)),
            pl.BlockSpec((Cout, 1), lambda n: (0, 0)),
        ],
        out_specs=pl.BlockSpec((1, H, W), lambda n: (n, 0, 0)),
        compiler_params=pltpu.CompilerParams(
            dimension_semantics=("parallel",),
            vmem_limit_bytes=64 * 1024 * 1024),
    )(x_t, w1_k, s1, b1, w2_k, b2_k)

    return out.reshape(N, Cout, H, W)


# speedup vs baseline: 1.1070x; 1.1070x over previous
import jax
import jax.numpy as jnp
from jax import lax
from jax.experimental import pallas as pl
from jax.experimental.pallas import tpu as pltpu


def _make_fused_kernel(H, W, Cin, Cmid, nb):
    HW = H * W

    def body(x_ref, w1_ref, sb_ref, w2_ref, b2_ref, out_ref):
        w1b = [w1_ref[k].astype(jnp.bfloat16) for k in range(3)]
        zrow = jnp.zeros((W, Cin), jnp.bfloat16)

        xx = lax.broadcasted_iota(jnp.int32, (1, HW), 1) % W
        ok_l = xx >= 1
        ok_r = xx <= W - 2

        s1 = sb_ref[:Cmid, :1]
        b1 = sb_ref[Cmid:, :1]

        dn_dims = (((1,), (1,)), ((), ()))
        for b in range(nb):
            x = x_ref[b].astype(jnp.bfloat16)

            up = jnp.concatenate([zrow, x[:-W]], axis=0)
            dn = jnp.concatenate([x[W:], zrow], axis=0)
            y = (lax.dot_general(w1b[0], up, dn_dims,
                                 preferred_element_type=jnp.float32)
                 + lax.dot_general(w1b[1], x, dn_dims,
                                   preferred_element_type=jnp.float32)
                 + lax.dot_general(w1b[2], dn, dn_dims,
                                   preferred_element_type=jnp.float32))

            mid = y[Cmid:2 * Cmid]
            lft = pltpu.roll(y[:Cmid], 1, 1)
            rgt = pltpu.roll(y[2 * Cmid:], HW - 1, 1)
            acc = (mid + jnp.where(ok_l, lft, 0.0)
                   + jnp.where(ok_r, rgt, 0.0))

            h = jnp.maximum(acc * s1 + b1, 0.0)

            out = jnp.dot(w2_ref[...], h, preferred_element_type=jnp.float32)
            out = out + b2_ref[...]
            out_ref[b] = out.reshape(H, W)

    return body


def kernel(x_nchw, w1, b1_conv, gamma, beta, mean, var, eps, w2, b2):
    N, Cin, H, W = x_nchw.shape
    Cmid = w1.shape[0]
    Cout = w2.shape[0]
    HW = H * W
    nb = 4 if N % 4 == 0 else (2 if N % 2 == 0 else 1)

    x_t = jnp.transpose(x_nchw, (0, 2, 3, 1)).reshape(N, HW, Cin)

    w1_k = jnp.transpose(w1, (2, 3, 0, 1)).reshape(3, 3 * Cmid, Cin)

    scale = gamma / jnp.sqrt(var + eps)
    bias = (b1_conv - mean) * scale + beta
    sb = jnp.broadcast_to(
        jnp.concatenate([scale, bias]).reshape(2 * Cmid, 1), (2 * Cmid, 128))

    w2_k = w2[:, :, 0, 0].astype(jnp.float32)
    b2_k = b2.reshape(Cout, 1).astype(jnp.float32)

    out = pl.pallas_call(
        _make_fused_kernel(H, W, Cin, Cmid, nb),
        out_shape=jax.ShapeDtypeStruct((N, H, W), jnp.float32),
        grid=(N // nb,),
        in_specs=[
            pl.BlockSpec((nb, HW, Cin), lambda n: (n, 0, 0)),
            pl.BlockSpec((3, 3 * Cmid, Cin), lambda n: (0, 0, 0)),
            pl.BlockSpec((2 * Cmid, 128), lambda n: (0, 0)),
            pl.BlockSpec((Cout, Cmid), lambda n: (0, 0)),
            pl.BlockSpec((Cout, 1), lambda n: (0, 0)),
        ],
        out_specs=pl.BlockSpec((nb, H, W), lambda n: (n, 0, 0)),
        compiler_params=pltpu.CompilerParams(
            dimension_semantics=("parallel",),
            vmem_limit_bytes=64 * 1024 * 1024),
    )(x_t, w1_k, sb, w2_k, b2_k)

    return out.reshape(N, Cout, H, W)
